# baseline (device time: 16001 ns/iter reference)
import jax
import jax.numpy as jnp
from jax import lax
from jax.experimental import pallas as pl
from jax.experimental.pallas import tpu as pltpu

N_DEV = 4
N_EXP = 8


def kernel(x, router_W, route_idx, expert_W):
    n_tok, d = x.shape
    e_per, _, h = expert_W.shape

    def body(x_ref, rw_ref, idx_ref, ew_ref, out_ref, comm_ref, send_sems, recv_sems):
        my_i = lax.axis_index("i")
        left = lax.rem(my_i - 1 + N_DEV, N_DEV)
        right = lax.rem(my_i + 1, N_DEV)

        barrier_sem = pltpu.get_barrier_semaphore()
        for nbr in [left, right]:
            pl.semaphore_signal(
                barrier_sem, inc=1,
                device_id=(nbr,), device_id_type=pl.DeviceIdType.MESH,
            )
        pl.semaphore_wait(barrier_sem, 2)

        x_f = x_ref[:, :]
        scores = jnp.dot(x_f, rw_ref[:, :], preferred_element_type=jnp.float32)
        s_max = jnp.max(scores, axis=-1, keepdims=True)
        p = jnp.exp(scores - s_max)
        probs = p / jnp.sum(p, axis=-1, keepdims=True)

        idx = idx_ref[:, :]
        e_iota = lax.broadcasted_iota(jnp.int32, (n_tok, N_EXP), 1)
        g0 = jnp.sum(jnp.where(e_iota == idx[:, 0:1], probs, 0.0), axis=1)
        g1 = jnp.sum(jnp.where(e_iota == idx[:, 1:2], probs, 0.0), axis=1)
        gs = g0 + g1
        g0 = g0 / gs
        g1 = g1 / gs

        partial = jnp.zeros((n_tok, h), jnp.float32)
        for le in range(e_per):
            e = my_i * e_per + le
            w = jnp.where(idx[:, 0] == e, g0, 0.0) + jnp.where(idx[:, 1] == e, g1, 0.0)
            xw = (x_f * w[:, None]).astype(jnp.bfloat16)
            partial = partial + jnp.dot(
                xw, ew_ref[le, :, :].astype(jnp.bfloat16),
                preferred_element_type=jnp.float32,
            )

        out_ref[:, :] = partial
        comm_ref[0, :, :] = partial.astype(jnp.bfloat16)

        for hop in range(N_DEV - 1):
            send_slot = hop % 2
            recv_slot = (hop + 1) % 2
            rdma = pltpu.make_async_remote_copy(
                src_ref=comm_ref.at[send_slot],
                dst_ref=comm_ref.at[recv_slot],
                send_sem=send_sems.at[send_slot],
                recv_sem=recv_sems.at[recv_slot],
                device_id=(right,),
                device_id_type=pl.DeviceIdType.MESH,
            )
            rdma.start()
            rdma.wait()
            out_ref[:, :] = out_ref[:, :] + comm_ref[recv_slot, :, :].astype(jnp.float32)

    return pl.pallas_call(
        body,
        out_shape=jax.ShapeDtypeStruct((n_tok, h), jnp.float32),
        in_specs=[
            pl.BlockSpec(memory_space=pltpu.VMEM),
            pl.BlockSpec(memory_space=pltpu.VMEM),
            pl.BlockSpec(memory_space=pltpu.VMEM),
            pl.BlockSpec(memory_space=pltpu.VMEM),
        ],
        out_specs=pl.BlockSpec(memory_space=pltpu.VMEM),
        scratch_shapes=[
            pltpu.VMEM((2, n_tok, h), jnp.bfloat16),
            pltpu.SemaphoreType.DMA((2,)),
            pltpu.SemaphoreType.DMA((2,)),
        ],
        compiler_params=pltpu.CompilerParams(collective_id=0),
    )(x, router_W, route_idx, expert_W)


# device time: 11262 ns/iter; 1.4208x vs baseline; 1.4208x over previous
import jax
import jax.numpy as jnp
from jax import lax
from jax.experimental import pallas as pl
from jax.experimental.pallas import tpu as pltpu

N_DEV = 4
N_EXP = 8


def kernel(x, router_W, route_idx, expert_W):
    n_tok, d = x.shape
    e_per, _, h = expert_W.shape

    def body(x_ref, rw_ref, idx_ref, ew_ref, out_ref, send_buf, recv_buf,
             send_sems, recv_sems):
        my_i = lax.axis_index("i")

        barrier_sem = pltpu.get_barrier_semaphore()
        for k in range(1, N_DEV):
            pl.semaphore_signal(
                barrier_sem, inc=1,
                device_id=(lax.rem(my_i + k, N_DEV),),
                device_id_type=pl.DeviceIdType.MESH,
            )
        pl.semaphore_wait(barrier_sem, N_DEV - 1)

        x_f = x_ref[:, :]
        scores = jnp.dot(x_f, rw_ref[:, :], preferred_element_type=jnp.float32)
        s_max = jnp.max(scores, axis=-1, keepdims=True)
        p = jnp.exp(scores - s_max)
        probs = p / jnp.sum(p, axis=-1, keepdims=True)

        idx = idx_ref[:, :]
        e_iota = lax.broadcasted_iota(jnp.int32, (n_tok, N_EXP), 1)
        g0 = jnp.sum(jnp.where(e_iota == idx[:, 0:1], probs, 0.0), axis=1)
        g1 = jnp.sum(jnp.where(e_iota == idx[:, 1:2], probs, 0.0), axis=1)
        gs = g0 + g1
        g0 = g0 / gs
        g1 = g1 / gs

        partial = jnp.zeros((n_tok, h), jnp.float32)
        for le in range(e_per):
            e = my_i * e_per + le
            w = jnp.where(idx[:, 0] == e, g0, 0.0) + jnp.where(idx[:, 1] == e, g1, 0.0)
            xw = (x_f * w[:, None]).astype(jnp.bfloat16)
            partial = partial + jnp.dot(
                xw, ew_ref[le, :, :].astype(jnp.bfloat16),
                preferred_element_type=jnp.float32,
            )

        out_ref[:, :] = partial
        send_buf[:, :] = partial.astype(jnp.bfloat16)

        rdmas = []
        for k in range(1, N_DEV):
            rdma = pltpu.make_async_remote_copy(
                src_ref=send_buf,
                dst_ref=recv_buf.at[k - 1],
                send_sem=send_sems.at[k - 1],
                recv_sem=recv_sems.at[k - 1],
                device_id=(lax.rem(my_i + k, N_DEV),),
                device_id_type=pl.DeviceIdType.MESH,
            )
            rdma.start()
            rdmas.append(rdma)

        for slot in (0, 2, 1):
            rdmas[slot].wait_recv()
            out_ref[:, :] = out_ref[:, :] + recv_buf[slot, :, :].astype(jnp.float32)
        for rdma in rdmas:
            rdma.wait_send()

    return pl.pallas_call(
        body,
        out_shape=jax.ShapeDtypeStruct((n_tok, h), jnp.float32),
        in_specs=[
            pl.BlockSpec(memory_space=pltpu.VMEM),
            pl.BlockSpec(memory_space=pltpu.VMEM),
            pl.BlockSpec(memory_space=pltpu.VMEM),
            pl.BlockSpec(memory_space=pltpu.VMEM),
        ],
        out_specs=pl.BlockSpec(memory_space=pltpu.VMEM),
        scratch_shapes=[
            pltpu.VMEM((n_tok, h), jnp.bfloat16),
            pltpu.VMEM((N_DEV - 1, n_tok, h), jnp.bfloat16),
            pltpu.SemaphoreType.DMA((N_DEV - 1,)),
            pltpu.SemaphoreType.DMA((N_DEV - 1,)),
        ],
        compiler_params=pltpu.CompilerParams(collective_id=0),
    )(x, router_W, route_idx, expert_W)


# device time: 10250 ns/iter; 1.5611x vs baseline; 1.0987x over previous
import jax
import jax.numpy as jnp
from jax import lax
from jax.experimental import pallas as pl
from jax.experimental.pallas import tpu as pltpu

N_DEV = 4
N_EXP = 8


def kernel(x, router_W, route_idx, expert_W):
    n_tok, d = x.shape
    e_per, _, h = expert_W.shape
    packed = jnp.concatenate([x, router_W.T], axis=0)

    def body(xp_ref, idx_ref, ew_ref, out_ref, send_buf, recv_buf,
             send_sems, recv_sems):
        my_i = lax.axis_index("i")

        barrier_sem = pltpu.get_barrier_semaphore()
        for k in range(1, N_DEV):
            pl.semaphore_signal(
                barrier_sem, inc=1,
                device_id=(lax.rem(my_i + k, N_DEV),),
                device_id_type=pl.DeviceIdType.MESH,
            )

        x_f = xp_ref[0:n_tok, :]
        rwt = xp_ref[n_tok:n_tok + N_EXP, :]
        scores = lax.dot_general(
            x_f, rwt, (((1,), (1,)), ((), ())),
            preferred_element_type=jnp.float32,
        )
        s_max = jnp.max(scores, axis=-1, keepdims=True)
        p = jnp.exp(scores - s_max)
        probs = p / jnp.sum(p, axis=-1, keepdims=True)

        idx = idx_ref[:, :]
        e_iota = lax.broadcasted_iota(jnp.int32, (n_tok, N_EXP), 1)
        g0 = jnp.sum(jnp.where(e_iota == idx[:, 0:1], probs, 0.0), axis=1)
        g1 = jnp.sum(jnp.where(e_iota == idx[:, 1:2], probs, 0.0), axis=1)
        gs = g0 + g1
        g0 = g0 / gs
        g1 = g1 / gs

        partial = jnp.zeros((n_tok, h), jnp.float32)
        for le in range(e_per):
            e = my_i * e_per + le
            w = jnp.where(idx[:, 0] == e, g0, 0.0) + jnp.where(idx[:, 1] == e, g1, 0.0)
            xw = (x_f * w[:, None]).astype(jnp.bfloat16)
            partial = partial + jnp.dot(
                xw, ew_ref[le, :, :].astype(jnp.bfloat16),
                preferred_element_type=jnp.float32,
            )

        send_buf[:, :] = partial.astype(jnp.bfloat16)
        pl.semaphore_wait(barrier_sem, N_DEV - 1)

        rdmas = {}
        for k in (2, 1, 3):
            rdma = pltpu.make_async_remote_copy(
                src_ref=send_buf,
                dst_ref=recv_buf.at[k - 1],
                send_sem=send_sems.at[k - 1],
                recv_sem=recv_sems.at[k - 1],
                device_id=(lax.rem(my_i + k, N_DEV),),
                device_id_type=pl.DeviceIdType.MESH,
            )
            rdma.start()
            rdmas[k - 1] = rdma

        acc = partial
        for slot in (0, 2, 1):
            rdmas[slot].wait_recv()
            acc = acc + recv_buf[slot, :, :].astype(jnp.float32)
        out_ref[:, :] = acc
        for rdma in rdmas.values():
            rdma.wait_send()

    return pl.pallas_call(
        body,
        out_shape=jax.ShapeDtypeStruct((n_tok, h), jnp.float32),
        in_specs=[
            pl.BlockSpec(memory_space=pltpu.VMEM),
            pl.BlockSpec(memory_space=pltpu.VMEM),
            pl.BlockSpec(memory_space=pltpu.VMEM),
        ],
        out_specs=pl.BlockSpec(memory_space=pltpu.VMEM),
        scratch_shapes=[
            pltpu.VMEM((n_tok, h), jnp.bfloat16),
            pltpu.VMEM((N_DEV - 1, n_tok, h), jnp.bfloat16),
            pltpu.SemaphoreType.DMA((N_DEV - 1,)),
            pltpu.SemaphoreType.DMA((N_DEV - 1,)),
        ],
        compiler_params=pltpu.CompilerParams(collective_id=0),
    )(packed, route_idx, expert_W)
